# baseline (speedup 1.0000x reference)
import jax
import jax.numpy as jnp
from jax import lax
from jax.experimental import pallas as pl
from jax.experimental.pallas import tpu as pltpu
from jax.experimental.pallas import tpu_sc as plsc

EMBED = 300
SEQ_LEN = 200
BATCH = 1024
HIDDEN = 256
NUM_CLASSES = 2
NUM_TABLES = 4

NUM_CORES = 2
NUM_SUBCORES = 16
NUM_WORKERS = NUM_CORES * NUM_SUBCORES
SPW = BATCH // NUM_WORKERS

LPAD = 256
TAIL = EMBED - 256
EPAD = 384
NUM_TCOLS = NUM_TABLES * EPAD // 128


def _sc_pool_body(bt_hbm, be_hbm, e_emb, t_emb, n2_emb, n3_emb,
                  tail_e, tail_t, tail_n2, tail_n3, out_hbm,
                  idx_t8, idx_e8, buf_a, buf_b, outb8, semg0, semg1):
  semg = (semg0, semg1)
  wid = lax.axis_index("s") * NUM_CORES + lax.axis_index("c")
  base = wid * SPW

  zero16 = jnp.zeros((16,), jnp.float32)
  def zbody(q, carry):
    for m in range(8):
      outb8[q // 8, q % 8, pl.ds(16 * m, 16)] = zero16
    return carry
  lax.fori_loop(0, NUM_TCOLS * 8, zbody, 0)

  def step_copy_args(st, s):
    sl = st % 2
    segs = (pl.ds(0, 128), pl.ds(128, 72))
    args = []
    if st < 4:
      tbl = (e_emb, t_emb, n2_emb, n3_emb)[st]
      idx = idx_e8 if st == 0 else idx_t8
      for j, bref in ((0, buf_a), (1, buf_b)):
        for seg in segs:
          args.append((tbl.at[idx.at[s, seg], pl.ds(128 * j, 128)],
                       bref.at[sl, seg, :], semg[sl]))
    elif st == 4:
      for tl, idx, bref in ((tail_e, idx_e8, buf_a), (tail_t, idx_t8, buf_b)):
        for seg in segs:
          args.append((tl.at[idx.at[s, seg], :], bref.at[sl, seg, :],
                       semg[sl]))
    else:
      for tl, bref in ((tail_n2, buf_a), (tail_n3, buf_b)):
        for seg in segs:
          args.append((tl.at[idx_t8.at[s, seg], :], bref.at[sl, seg, :],
                       semg[sl]))
    return args

  def issue_step(st, s):
    for a in step_copy_args(st, s):
      pltpu.async_copy(*a)

  def wait_step(st, s):
    for a in step_copy_args(st, s):
      pltpu.make_async_copy(*a).wait()

  def reduce_chunks(bref, sl, cols, dsts, s):
    def rbody(r, accs):
      return tuple(acc + bref[sl, r, pl.ds(c, 16)]
                   for acc, c in zip(accs, cols))
    init = tuple(bref[sl, 0, pl.ds(c, 16)] for c in cols)
    accs = lax.fori_loop(1, SEQ_LEN, rbody, init)
    for acc, (tc, w) in zip(accs, dsts):
      outb8[tc, s, pl.ds(w, 16)] = acc

  def reduce_step(st, s):
    sl = st % 2
    if st < 4:
      cols = tuple(16 * m for m in range(8))
      reduce_chunks(buf_a, sl, cols,
                    tuple((3 * st, 16 * m) for m in range(8)), s)
      reduce_chunks(buf_b, sl, cols,
                    tuple((3 * st + 1, 16 * m) for m in range(8)), s)
    else:
      ta, tb = (2, 5) if st == 4 else (8, 11)
      reduce_chunks(buf_a, sl, (0, 16, 28),
                    ((ta, 0), (ta, 16), (ta, 28)), s)
      reduce_chunks(buf_b, sl, (0, 16, 28),
                    ((tb, 0), (tb, 16), (tb, 28)), s)

  def do_sample(i, carry):
    s = lax.rem(i, 8)
    b = base + i

    @pl.when(s == 0)
    def _():
      bg = pl.multiple_of(b, 8)
      pltpu.sync_copy(bt_hbm.at[pl.ds(bg, 8)], idx_t8)
      pltpu.sync_copy(be_hbm.at[pl.ds(bg, 8)], idx_e8)
      issue_step(0, s)

    for st in range(6):
      if st < 5:
        issue_step(st + 1, s)
      else:
        @pl.when(s < 7)
        def _():
          issue_step(0, s + 1)
      wait_step(st, s)
      reduce_step(st, s)

    @pl.when(s == 7)
    def _():
      grp = (b - 7) // 8
      pltpu.sync_copy(outb8, out_hbm.at[grp])
    return carry

  lax.fori_loop(0, SPW, do_sample, 0)


def _make_sc_pool():
  mesh = plsc.VectorSubcoreMesh(core_axis_name="c", subcore_axis_name="s",
                                num_cores=NUM_CORES,
                                num_subcores=NUM_SUBCORES)
  return pl.kernel(
      _sc_pool_body,
      out_type=jax.ShapeDtypeStruct((BATCH // 8, NUM_TCOLS, 8, 128),
                                    jnp.float32),
      mesh=mesh,
      scratch_types=[
          pltpu.VMEM((8, LPAD), jnp.int32),
          pltpu.VMEM((8, LPAD), jnp.int32),
          pltpu.VMEM((2, SEQ_LEN, 128), jnp.float32),
          pltpu.VMEM((2, SEQ_LEN, 128), jnp.float32),
          pltpu.VMEM((NUM_TCOLS, 8, 128), jnp.float32),
          pltpu.SemaphoreType.DMA,
          pltpu.SemaphoreType.DMA,
      ],
  )


def _tailpack_body(x_ref, y_ref, ox_ref, oy_ref):
  ox_ref[...] = x_ref[...]
  oy_ref[...] = y_ref[...]


def _tailpack(x, y):
  v = x.shape[0]
  rb = 2048
  grid = (v + rb - 1) // rb
  in_spec = pl.BlockSpec((rb, 128), lambda i: (i, 2))
  out_spec = pl.BlockSpec((rb, 128), lambda i: (i, 0))
  return pl.pallas_call(
      _tailpack_body,
      grid=(grid,),
      in_specs=[in_spec, in_spec],
      out_specs=[out_spec, out_spec],
      out_shape=[jax.ShapeDtypeStruct((v, 128), jnp.float32)] * 2,
  )(x, y)


def _mlp_body(acc_ref, w1_ref, b1_ref, w2_ref, b2_ref, out_ref):
  acc = acc_ref[...]
  h = lax.dot_general(acc, w1_ref[...], (((1,), (0,)), ((), ())),
                      preferred_element_type=jnp.float32,
                      precision=lax.Precision.HIGHEST)
  h = h * (1.0 / SEQ_LEN) + b1_ref[...]
  h = jnp.maximum(h, 0.0)
  logits = lax.dot_general(h, w2_ref[...], (((1,), (0,)), ((), ())),
                           preferred_element_type=jnp.float32,
                           precision=lax.Precision.HIGHEST) + b2_ref[...]
  m = jnp.max(logits, axis=1, keepdims=True)
  lse = jnp.log(jnp.sum(jnp.exp(logits - m), axis=1, keepdims=True)) + m
  out_ref[...] = logits - lse


def kernel(text, entity1, text_emb, entity_emb, ngram2_emb, ngram3_emb,
           W1, b1, W2, b2):
  bt = jnp.pad(text.T, ((0, 0), (0, LPAD - SEQ_LEN)))
  be = jnp.pad(entity1.T, ((0, 0), (0, LPAD - SEQ_LEN)))

  tl_t, tl_e = _tailpack(text_emb, entity_emb)
  tl_n2, tl_n3 = _tailpack(ngram2_emb, ngram3_emb)

  acc4 = _make_sc_pool()(bt, be, entity_emb, text_emb, ngram2_emb,
                         ngram3_emb, tl_e, tl_t, tl_n2, tl_n3)
  acc = acc4.reshape(BATCH, NUM_TABLES * EPAD)

  w1r = W1.reshape(NUM_TABLES, EMBED, HIDDEN)
  w1p = jnp.zeros((NUM_TABLES, EPAD, HIDDEN), jnp.float32)
  w1p = w1p.at[:, :EMBED, :].set(w1r)
  w1p = w1p.reshape(NUM_TABLES * EPAD, HIDDEN)

  return pl.pallas_call(
      _mlp_body,
      out_shape=jax.ShapeDtypeStruct((BATCH, NUM_CLASSES), jnp.float32),
  )(acc, w1p, b1.reshape(1, HIDDEN), W2, b2.reshape(1, NUM_CLASSES))


# speedup vs baseline: 1.2262x; 1.2262x over previous
import jax
import jax.numpy as jnp
from jax import lax
from jax.experimental import pallas as pl
from jax.experimental.pallas import tpu as pltpu
from jax.experimental.pallas import tpu_sc as plsc

VOCAB = 100000
EMBED = 300
SEQ_LEN = 200
BATCH = 1024
HIDDEN = 256
NUM_CLASSES = 2
NUM_TABLES = 4

NUM_CORES = 2
NUM_SUBCORES = 16
NUM_WORKERS = NUM_CORES * NUM_SUBCORES
SPW = BATCH // NUM_WORKERS

LPAD = 256
TAIL = EMBED - 256
EPAD = 384
NUM_TCOLS = NUM_TABLES * EPAD // 128


def _sc_pool_body(bt_hbm, be_hbm, e_emb, t_emb, n2_emb, n3_emb,
                  tail_e, tail_t, tail_n2, tail_n3, out_hbm,
                  idx_t8, idx_e8, buf_a, buf_b, outb8, semg0, semg1):
  semg = (semg0, semg1)
  wid = lax.axis_index("s") * NUM_CORES + lax.axis_index("c")
  base = wid * SPW

  zero16 = jnp.zeros((16,), jnp.float32)
  def zbody(q, carry):
    for m in range(8):
      outb8[q // 8, q % 8, pl.ds(16 * m, 16)] = zero16
    return carry
  lax.fori_loop(0, NUM_TCOLS * 8, zbody, 0)

  def step_copy_args(st, s):
    sl = st % 2
    segs = (pl.ds(0, 128), pl.ds(128, 72))
    args = []
    if st < 4:
      tbl = (e_emb, t_emb, n2_emb, n3_emb)[st]
      idx = idx_e8 if st == 0 else idx_t8
      for j, bref in ((0, buf_a), (1, buf_b)):
        for seg in segs:
          args.append((tbl.at[idx.at[s, seg], pl.ds(128 * j, 128)],
                       bref.at[sl, seg, :], semg[sl]))
    elif st == 4:
      for tl, idx, bref in ((tail_e, idx_e8, buf_a), (tail_t, idx_t8, buf_b)):
        for seg in segs:
          args.append((tl.at[idx.at[s, seg], :], bref.at[sl, seg, :],
                       semg[sl]))
    else:
      for tl, bref in ((tail_n2, buf_a), (tail_n3, buf_b)):
        for seg in segs:
          args.append((tl.at[idx_t8.at[s, seg], :], bref.at[sl, seg, :],
                       semg[sl]))
    return args

  def issue_step(st, s):
    for a in step_copy_args(st, s):
      pltpu.async_copy(*a)

  def wait_step(st, s):
    for a in step_copy_args(st, s):
      pltpu.make_async_copy(*a).wait()

  def reduce_chunks(bref, sl, cols, dsts, s):
    def rbody(r, accs):
      return tuple(acc + bref[sl, r, pl.ds(c, 16)]
                   for acc, c in zip(accs, cols))
    init = tuple(bref[sl, 0, pl.ds(c, 16)] for c in cols)
    accs = lax.fori_loop(1, SEQ_LEN, rbody, init)
    for acc, (tc, w) in zip(accs, dsts):
      outb8[tc, s, pl.ds(w, 16)] = acc

  def reduce_step(st, s):
    sl = st % 2
    if st < 4:
      cols = tuple(16 * m for m in range(8))
      reduce_chunks(buf_a, sl, cols,
                    tuple((3 * st, 16 * m) for m in range(8)), s)
      reduce_chunks(buf_b, sl, cols,
                    tuple((3 * st + 1, 16 * m) for m in range(8)), s)
    else:
      ta, tb = (2, 5) if st == 4 else (8, 11)
      reduce_chunks(buf_a, sl, (0, 16, 28),
                    ((ta, 0), (ta, 16), (ta, 28)), s)
      reduce_chunks(buf_b, sl, (0, 16, 28),
                    ((tb, 0), (tb, 16), (tb, 28)), s)

  def do_sample(i, carry):
    s = lax.rem(i, 8)
    b = base + i

    @pl.when(s == 0)
    def _():
      bg = pl.multiple_of(b, 8)
      pltpu.sync_copy(bt_hbm.at[pl.ds(bg, 8)], idx_t8)
      pltpu.sync_copy(be_hbm.at[pl.ds(bg, 8)], idx_e8)
      issue_step(0, s)

    for st in range(6):
      if st < 5:
        issue_step(st + 1, s)
      else:
        @pl.when(s < 7)
        def _():
          issue_step(0, s + 1)
      wait_step(st, s)
      reduce_step(st, s)

    @pl.when(s == 7)
    def _():
      grp = (b - 7) // 8
      pltpu.sync_copy(outb8, out_hbm.at[grp])
    return carry

  lax.fori_loop(0, SPW, do_sample, 0)


def _make_sc_pool():
  mesh = plsc.VectorSubcoreMesh(core_axis_name="c", subcore_axis_name="s",
                                num_cores=NUM_CORES,
                                num_subcores=NUM_SUBCORES)
  return pl.kernel(
      _sc_pool_body,
      out_type=jax.ShapeDtypeStruct((BATCH // 8, NUM_TCOLS, 8, 128),
                                    jnp.float32),
      mesh=mesh,
      scratch_types=[
          pltpu.VMEM((8, LPAD), jnp.int32),
          pltpu.VMEM((8, LPAD), jnp.int32),
          pltpu.VMEM((2, SEQ_LEN, 128), jnp.float32),
          pltpu.VMEM((2, SEQ_LEN, 128), jnp.float32),
          pltpu.VMEM((NUM_TCOLS, 8, 128), jnp.float32),
          pltpu.SemaphoreType.DMA,
          pltpu.SemaphoreType.DMA,
      ],
  )


def _tailpack_body(x_ref, y_ref, ox_ref, oy_ref):
  ox_ref[...] = x_ref[...]
  oy_ref[...] = y_ref[...]


def _tailpack(x, y):
  v = x.shape[0]
  rb = 2048
  grid = (v + rb - 1) // rb
  in_spec = pl.BlockSpec((rb, 128), lambda i: (i, 2))
  out_spec = pl.BlockSpec((rb, 128), lambda i: (i, 0))
  return pl.pallas_call(
      _tailpack_body,
      grid=(grid,),
      in_specs=[in_spec, in_spec],
      out_specs=[out_spec, out_spec],
      out_shape=[jax.ShapeDtypeStruct((v, 128), jnp.float32)] * 2,
  )(x, y)


def _mlp_body(acc_ref, w1_ref, b1_ref, w2_ref, b2_ref, out_ref):
  acc = acc_ref[...]
  h = lax.dot_general(acc, w1_ref[...], (((1,), (0,)), ((), ())),
                      preferred_element_type=jnp.float32,
                      precision=lax.Precision.HIGHEST)
  h = h * (1.0 / SEQ_LEN) + b1_ref[...]
  h = jnp.maximum(h, 0.0)
  logits = lax.dot_general(h, w2_ref[...], (((1,), (0,)), ((), ())),
                           preferred_element_type=jnp.float32,
                           precision=lax.Precision.HIGHEST) + b2_ref[...]
  m = jnp.max(logits, axis=1, keepdims=True)
  lse = jnp.log(jnp.sum(jnp.exp(logits - m), axis=1, keepdims=True)) + m
  out_ref[...] = logits - lse


def kernel(text, entity1, text_emb, entity_emb, ngram2_emb, ngram3_emb,
           W1, b1, W2, b2):
  bt = jnp.pad(text.T, ((0, 0), (0, LPAD - SEQ_LEN)))
  be = jnp.pad(entity1.T, ((0, 0), (0, LPAD - SEQ_LEN)))

  n2s = ngram2_emb[:VOCAB]
  n3s = ngram3_emb[:VOCAB]

  tl_t, tl_e = _tailpack(text_emb, entity_emb)
  tl_n2, tl_n3 = _tailpack(n2s, n3s)

  acc4 = _make_sc_pool()(bt, be, entity_emb, text_emb, n2s, n3s,
                         tl_e, tl_t, tl_n2, tl_n3)
  acc = acc4.reshape(BATCH, NUM_TABLES * EPAD)

  w1r = W1.reshape(NUM_TABLES, EMBED, HIDDEN)
  w1p = jnp.zeros((NUM_TABLES, EPAD, HIDDEN), jnp.float32)
  w1p = w1p.at[:, :EMBED, :].set(w1r)
  w1p = w1p.reshape(NUM_TABLES * EPAD, HIDDEN)

  return pl.pallas_call(
      _mlp_body,
      out_shape=jax.ShapeDtypeStruct((BATCH, NUM_CLASSES), jnp.float32),
  )(acc, w1p, b1.reshape(1, HIDDEN), W2, b2.reshape(1, NUM_CLASSES))


# speedup vs baseline: 1.3720x; 1.1189x over previous
import jax
import jax.numpy as jnp
from jax import lax
from jax.experimental import pallas as pl
from jax.experimental.pallas import tpu as pltpu
from jax.experimental.pallas import tpu_sc as plsc

VOCAB = 100000
EMBED = 300
SEQ_LEN = 200
BATCH = 1024
HIDDEN = 256
NUM_CLASSES = 2
NUM_TABLES = 4

NUM_CORES = 2
NUM_SUBCORES = 16
NUM_WORKERS = NUM_CORES * NUM_SUBCORES
SPW = BATCH // NUM_WORKERS

LPAD = 256
TAIL = EMBED - 256
EPAD = 384


def _sc_table_body(idx_hbm, tbl, tail, out_hbm, idx8, buf, outb8,
                   semg0, semg1):
  semg = (semg0, semg1)
  wid = lax.axis_index("s") * NUM_CORES + lax.axis_index("c")
  base = wid * SPW

  zero16 = jnp.zeros((16,), jnp.float32)
  def zbody(q, carry):
    for m in range(8):
      outb8[q // 8, q % 8, pl.ds(16 * m, 16)] = zero16
    return carry
  lax.fori_loop(0, 3 * 8, zbody, 0)

  segs = (pl.ds(0, 128), pl.ds(128, 72))

  def step_copy_args(st, s, sl):
    if st < 2:
      return [(tbl.at[idx8.at[s, seg], pl.ds(128 * st, 128)],
               buf.at[sl, seg, :], semg[sl]) for seg in segs]
    return [(tail.at[idx8.at[s, seg], :], buf.at[sl, seg, :], semg[sl])
            for seg in segs]

  def issue_step(st, s, sl):
    for a in step_copy_args(st, s, sl):
      pltpu.async_copy(*a)

  def wait_step(st, s, sl):
    for a in step_copy_args(st, s, sl):
      pltpu.make_async_copy(*a).wait()

  def reduce_step(st, s, sl):
    cols = tuple(16 * m for m in range(8)) if st < 2 else (0, 16, 28)
    def rbody(r, accs):
      return tuple(acc + buf[sl, r, pl.ds(c, 16)]
                   for acc, c in zip(accs, cols))
    init = tuple(buf[sl, 0, pl.ds(c, 16)] for c in cols)
    accs = lax.fori_loop(1, SEQ_LEN, rbody, init)
    for acc, c in zip(accs, cols):
      outb8[st, s, pl.ds(c, 16)] = acc

  def do_pair(k, carry):
    i0 = 2 * k
    s0 = lax.rem(i0, 8)
    b0 = base + i0

    @pl.when(s0 == 0)
    def _():
      bg = pl.multiple_of(b0, 8)
      pltpu.sync_copy(idx_hbm.at[pl.ds(bg, 8)], idx8)
      issue_step(0, s0, 0)

    for half in range(2):
      s = s0 + half
      for st in range(3):
        sl = (3 * half + st) % 2
        nsl = 1 - sl
        if st < 2:
          issue_step(st + 1, s, nsl)
        elif half == 0:
          issue_step(0, s + 1, nsl)
        else:
          @pl.when(lax.rem(s, 8) < 7)
          def _():
            issue_step(0, s + 1, nsl)
        wait_step(st, s, sl)
        reduce_step(st, s, sl)

      @pl.when(lax.rem(s, 8) == 7)
      def _():
        grp = (base + i0 + half - 7) // 8
        pltpu.sync_copy(outb8, out_hbm.at[grp])
    return carry

  lax.fori_loop(0, SPW // 2, do_pair, 0)


def _make_sc_pool(vocab):
  mesh = plsc.VectorSubcoreMesh(core_axis_name="c", subcore_axis_name="s",
                                num_cores=NUM_CORES,
                                num_subcores=NUM_SUBCORES)
  return pl.kernel(
      _sc_table_body,
      out_type=jax.ShapeDtypeStruct((BATCH // 8, 3, 8, 128), jnp.float32),
      mesh=mesh,
      scratch_types=[
          pltpu.VMEM((8, LPAD), jnp.int32),
          pltpu.VMEM((2, SEQ_LEN, 128), jnp.float32),
          pltpu.VMEM((3, 8, 128), jnp.float32),
          pltpu.SemaphoreType.DMA,
          pltpu.SemaphoreType.DMA,
      ],
  )


def _tailpack_body(x_ref, y_ref, ox_ref, oy_ref):
  ox_ref[...] = x_ref[...]
  oy_ref[...] = y_ref[...]


def _tailpack(x, y):
  v = x.shape[0]
  rb = 2048
  grid = (v + rb - 1) // rb
  in_spec = pl.BlockSpec((rb, 128), lambda i: (i, 2))
  out_spec = pl.BlockSpec((rb, 128), lambda i: (i, 0))
  return pl.pallas_call(
      _tailpack_body,
      grid=(grid,),
      in_specs=[in_spec, in_spec],
      out_specs=[out_spec, out_spec],
      out_shape=[jax.ShapeDtypeStruct((v, 128), jnp.float32)] * 2,
  )(x, y)


def _mlp_body(a0, a1, a2, a3, w1_ref, b1_ref, w2_ref, b2_ref, out_ref):
  h = None
  for t, a in enumerate((a0, a1, a2, a3)):
    p = lax.dot_general(a[...], w1_ref[pl.ds(EPAD * t, EPAD), :],
                        (((1,), (0,)), ((), ())),
                        preferred_element_type=jnp.float32,
                        precision=lax.Precision.HIGHEST)
    h = p if h is None else h + p
  h = h * (1.0 / SEQ_LEN) + b1_ref[...]
  h = jnp.maximum(h, 0.0)
  logits = lax.dot_general(h, w2_ref[...], (((1,), (0,)), ((), ())),
                           preferred_element_type=jnp.float32,
                           precision=lax.Precision.HIGHEST) + b2_ref[...]
  m = jnp.max(logits, axis=1, keepdims=True)
  lse = jnp.log(jnp.sum(jnp.exp(logits - m), axis=1, keepdims=True)) + m
  out_ref[...] = logits - lse


def kernel(text, entity1, text_emb, entity_emb, ngram2_emb, ngram3_emb,
           W1, b1, W2, b2):
  bt = jnp.pad(text.T, ((0, 0), (0, LPAD - SEQ_LEN)))
  be = jnp.pad(entity1.T, ((0, 0), (0, LPAD - SEQ_LEN)))

  n2s = ngram2_emb[:VOCAB]
  n3s = ngram3_emb[:VOCAB]

  tl_t, tl_e = _tailpack(text_emb, entity_emb)
  tl_n2, tl_n3 = _tailpack(n2s, n3s)

  pool = _make_sc_pool(VOCAB)
  acc_e = pool(be, entity_emb, tl_e)
  acc_t = pool(bt, text_emb, tl_t)
  acc_n2 = pool(bt, n2s, tl_n2)
  acc_n3 = pool(bt, n3s, tl_n3)
  accs = [a.reshape(BATCH, EPAD) for a in (acc_e, acc_t, acc_n2, acc_n3)]

  w1r = W1.reshape(NUM_TABLES, EMBED, HIDDEN)
  w1p = jnp.zeros((NUM_TABLES, EPAD, HIDDEN), jnp.float32)
  w1p = w1p.at[:, :EMBED, :].set(w1r)
  w1p = w1p.reshape(NUM_TABLES * EPAD, HIDDEN)

  return pl.pallas_call(
      _mlp_body,
      out_shape=jax.ShapeDtypeStruct((BATCH, NUM_CLASSES), jnp.float32),
  )(*accs, w1p, b1.reshape(1, HIDDEN), W2, b2.reshape(1, NUM_CLASSES))


# speedup vs baseline: 1.6805x; 1.2249x over previous
import jax
import jax.numpy as jnp
from jax import lax
from jax.experimental import pallas as pl
from jax.experimental.pallas import tpu as pltpu
from jax.experimental.pallas import tpu_sc as plsc

VOCAB = 100000
EMBED = 300
SEQ_LEN = 200
BATCH = 1024
HIDDEN = 256
NUM_CLASSES = 2
NUM_TABLES = 4

NUM_CORES = 2
NUM_SUBCORES = 16
NUM_WORKERS = NUM_CORES * NUM_SUBCORES
SPW = BATCH // NUM_WORKERS

LPAD = 256
TAIL = EMBED - 256
EPAD = 384


def _sc_table_body(idx_hbm, tbl, tail, out_hbm, idx8, buf, outb8,
                   semg0, semg1):
  semg = (semg0, semg1)
  wid = lax.axis_index("s") * NUM_CORES + lax.axis_index("c")
  base = wid * SPW

  zero16 = jnp.zeros((16,), jnp.float32)
  def zbody(q, carry):
    for m in range(8):
      outb8[q // 8, q % 8, pl.ds(16 * m, 16)] = zero16
    return carry
  lax.fori_loop(0, 3 * 8, zbody, 0)

  segs = (pl.ds(0, 128), pl.ds(128, 72))

  def step_copy_args(st, s, sl):
    if st < 2:
      return [(tbl.at[idx8.at[s, seg], pl.ds(128 * st, 128)],
               buf.at[sl, seg, :], semg[sl]) for seg in segs]
    return [(tail.at[idx8.at[s, seg], :], buf.at[sl, seg, :], semg[sl])
            for seg in segs]

  def issue_step(st, s, sl):
    for a in step_copy_args(st, s, sl):
      pltpu.async_copy(*a)

  def wait_step(st, s, sl):
    for a in step_copy_args(st, s, sl):
      pltpu.make_async_copy(*a).wait()

  def reduce_step(st, s, sl):
    cols = tuple(16 * m for m in range(8)) if st < 2 else (0, 16, 28)
    def rbody(r, accs):
      return tuple(acc + buf[sl, r, pl.ds(c, 16)]
                   for acc, c in zip(accs, cols))
    init = tuple(buf[sl, 0, pl.ds(c, 16)] for c in cols)
    accs = lax.fori_loop(1, SEQ_LEN, rbody, init)
    for acc, c in zip(accs, cols):
      outb8[st, s, pl.ds(c, 16)] = acc

  def do_pair(k, carry):
    i0 = 2 * k
    s0 = lax.rem(i0, 8)
    b0 = base + i0

    @pl.when(s0 == 0)
    def _():
      bg = pl.multiple_of(b0, 8)
      pltpu.sync_copy(idx_hbm.at[pl.ds(bg, 8)], idx8)
      issue_step(0, s0, 0)

    for half in range(2):
      s = s0 + half
      for st in range(3):
        sl = (3 * half + st) % 2
        nsl = 1 - sl
        if st < 2:
          issue_step(st + 1, s, nsl)
        elif half == 0:
          issue_step(0, s + 1, nsl)
        else:
          @pl.when(lax.rem(s, 8) < 7)
          def _():
            issue_step(0, s + 1, nsl)
        wait_step(st, s, sl)
        reduce_step(st, s, sl)

      @pl.when(lax.rem(s, 8) == 7)
      def _():
        grp = (base + i0 + half - 7) // 8
        pltpu.sync_copy(outb8, out_hbm.at[grp])
    return carry

  lax.fori_loop(0, SPW // 2, do_pair, 0)


def _make_sc_pool(vocab):
  mesh = plsc.VectorSubcoreMesh(core_axis_name="c", subcore_axis_name="s",
                                num_cores=NUM_CORES,
                                num_subcores=NUM_SUBCORES)
  return pl.kernel(
      _sc_table_body,
      out_type=jax.ShapeDtypeStruct((BATCH // 8, 3, 8, 128), jnp.float32),
      mesh=mesh,
      scratch_types=[
          pltpu.VMEM((8, LPAD), jnp.int32),
          pltpu.VMEM((2, SEQ_LEN, 128), jnp.float32),
          pltpu.VMEM((3, 8, 128), jnp.float32),
          pltpu.SemaphoreType.DMA,
          pltpu.SemaphoreType.DMA,
      ],
  )


RB = 512


def _relayout_body(xt_ref, om_ref, ot_ref):
  x = xt_ref[...].T
  om_ref[...] = x
  ot_ref[:, :TAIL] = x[:, 256:]


def _relayout(xt):
  grid = ((VOCAB + RB - 1) // RB,)
  return pl.pallas_call(
      _relayout_body,
      grid=grid,
      in_specs=[pl.BlockSpec((EMBED, RB), lambda i: (0, i))],
      out_specs=[pl.BlockSpec((RB, EMBED), lambda i: (i, 0)),
                 pl.BlockSpec((RB, 128), lambda i: (i, 0))],
      out_shape=[jax.ShapeDtypeStruct((VOCAB, EMBED), jnp.float32),
                 jax.ShapeDtypeStruct((VOCAB, 128), jnp.float32)],
  )(xt)


def _mlp_body(a0, a1, a2, a3, w1_ref, b1_ref, w2_ref, b2_ref, out_ref):
  h = None
  for t, a in enumerate((a0, a1, a2, a3)):
    p = lax.dot_general(a[...], w1_ref[pl.ds(EPAD * t, EPAD), :],
                        (((1,), (0,)), ((), ())),
                        preferred_element_type=jnp.float32,
                        precision=lax.Precision.HIGHEST)
    h = p if h is None else h + p
  h = h * (1.0 / SEQ_LEN) + b1_ref[...]
  h = jnp.maximum(h, 0.0)
  logits = lax.dot_general(h, w2_ref[...], (((1,), (0,)), ((), ())),
                           preferred_element_type=jnp.float32,
                           precision=lax.Precision.HIGHEST) + b2_ref[...]
  m = jnp.max(logits, axis=1, keepdims=True)
  lse = jnp.log(jnp.sum(jnp.exp(logits - m), axis=1, keepdims=True)) + m
  out_ref[...] = logits - lse


def kernel(text, entity1, text_emb, entity_emb, ngram2_emb, ngram3_emb,
           W1, b1, W2, b2):
  bt = jnp.pad(text.T, ((0, 0), (0, LPAD - SEQ_LEN)))
  be = jnp.pad(entity1.T, ((0, 0), (0, LPAD - SEQ_LEN)))

  m_e, tl_e = _relayout(entity_emb.T)
  t_gate, _ = lax.optimization_barrier((text_emb, tl_e))
  m_t, tl_t = _relayout(t_gate.T)
  n2_gate, _ = lax.optimization_barrier((ngram2_emb, tl_t))
  m_n2, tl_n2 = _relayout(n2_gate.T)
  n3_gate, _ = lax.optimization_barrier((ngram3_emb, tl_n2))
  m_n3, tl_n3 = _relayout(n3_gate.T)

  pool = _make_sc_pool(VOCAB)
  acc_e = pool(be, m_e, tl_e)
  acc_t = pool(bt, m_t, tl_t)
  acc_n2 = pool(bt, m_n2, tl_n2)
  acc_n3 = pool(bt, m_n3, tl_n3)
  accs = [a.reshape(BATCH, EPAD) for a in (acc_e, acc_t, acc_n2, acc_n3)]

  w1r = W1.reshape(NUM_TABLES, EMBED, HIDDEN)
  w1p = jnp.zeros((NUM_TABLES, EPAD, HIDDEN), jnp.float32)
  w1p = w1p.at[:, :EMBED, :].set(w1r)
  w1p = w1p.reshape(NUM_TABLES * EPAD, HIDDEN)

  return pl.pallas_call(
      _mlp_body,
      out_shape=jax.ShapeDtypeStruct((BATCH, NUM_CLASSES), jnp.float32),
  )(*accs, w1p, b1.reshape(1, HIDDEN), W2, b2.reshape(1, NUM_CLASSES))


# speedup vs baseline: 1.7811x; 1.0599x over previous
import jax
import jax.numpy as jnp
from jax import lax
from jax.experimental import pallas as pl
from jax.experimental.pallas import tpu as pltpu
from jax.experimental.pallas import tpu_sc as plsc

VOCAB = 100000
EMBED = 300
SEQ_LEN = 200
BATCH = 1024
HIDDEN = 256
NUM_CLASSES = 2
NUM_TABLES = 4

NUM_CORES = 2
NUM_SUBCORES = 16
NUM_WORKERS = NUM_CORES * NUM_SUBCORES
SPW = BATCH // NUM_WORKERS

LPAD = 256
TAIL = EMBED - 256
EPAD = 384


def _sc_table_body(idx_hbm, tbl, tail, out_hbm, idx8, buf, outb8,
                   semg0, semg1):
  semg = (semg0, semg1)
  wid = lax.axis_index("s") * NUM_CORES + lax.axis_index("c")
  base = wid * SPW

  zero16 = jnp.zeros((16,), jnp.float32)
  def zbody(q, carry):
    for m in range(8):
      outb8[q // 8, q % 8, pl.ds(16 * m, 16)] = zero16
    return carry
  lax.fori_loop(0, 3 * 8, zbody, 0)

  segs = (pl.ds(0, 128), pl.ds(128, 72))

  def step_copy_args(st, s, sl):
    if st < 2:
      return [(tbl.at[idx8.at[s, seg], pl.ds(128 * st, 128)],
               buf.at[sl, seg, :], semg[sl]) for seg in segs]
    return [(tail.at[idx8.at[s, seg], :], buf.at[sl, seg, :], semg[sl])
            for seg in segs]

  def issue_step(st, s, sl):
    for a in step_copy_args(st, s, sl):
      pltpu.async_copy(*a)

  def wait_step(st, s, sl):
    for a in step_copy_args(st, s, sl):
      pltpu.make_async_copy(*a).wait()

  def reduce_step(st, s, sl):
    cols = tuple(16 * m for m in range(8)) if st < 2 else (0, 16, 28)
    def rbody(r, accs):
      return tuple(acc + buf[sl, r, pl.ds(c, 16)]
                   for acc, c in zip(accs, cols))
    init = tuple(buf[sl, 0, pl.ds(c, 16)] for c in cols)
    accs = lax.fori_loop(1, SEQ_LEN, rbody, init)
    for acc, c in zip(accs, cols):
      outb8[st, s, pl.ds(c, 16)] = acc

  def do_pair(k, carry):
    i0 = 2 * k
    s0 = lax.rem(i0, 8)
    b0 = base + i0

    @pl.when(s0 == 0)
    def _():
      bg = pl.multiple_of(b0, 8)
      pltpu.sync_copy(idx_hbm.at[pl.ds(bg, 8)], idx8)
      issue_step(0, s0, 0)

    for half in range(2):
      s = s0 + half
      for st in range(3):
        sl = (3 * half + st) % 2
        nsl = 1 - sl
        if st < 2:
          issue_step(st + 1, s, nsl)
        elif half == 0:
          issue_step(0, s + 1, nsl)
        else:
          @pl.when(lax.rem(s, 8) < 7)
          def _():
            issue_step(0, s + 1, nsl)
        wait_step(st, s, sl)
        reduce_step(st, s, sl)

      @pl.when(lax.rem(s, 8) == 7)
      def _():
        grp = (base + i0 + half - 7) // 8
        pltpu.sync_copy(outb8, out_hbm.at[grp])
    return carry

  lax.fori_loop(0, SPW // 2, do_pair, 0)


def _make_sc_pool(vocab):
  mesh = plsc.VectorSubcoreMesh(core_axis_name="c", subcore_axis_name="s",
                                num_cores=NUM_CORES,
                                num_subcores=NUM_SUBCORES)
  return pl.kernel(
      _sc_table_body,
      out_type=jax.ShapeDtypeStruct((BATCH // 8, 3, 8, 128), jnp.float32),
      mesh=mesh,
      scratch_types=[
          pltpu.VMEM((8, LPAD), jnp.int32),
          pltpu.VMEM((2, SEQ_LEN, 128), jnp.float32),
          pltpu.VMEM((3, 8, 128), jnp.float32),
          pltpu.SemaphoreType.DMA,
          pltpu.SemaphoreType.DMA,
      ],
  )


RB = 512


def _relayout_body(xt_ref, om_ref, ot_ref):
  x = xt_ref[...].T
  om_ref[...] = x[:, :256]
  ot_ref[:, :TAIL] = x[:, 256:]


def _relayout(xt):
  grid = ((VOCAB + RB - 1) // RB,)
  return pl.pallas_call(
      _relayout_body,
      grid=grid,
      in_specs=[pl.BlockSpec((EMBED, RB), lambda i: (0, i))],
      out_specs=[pl.BlockSpec((RB, 256), lambda i: (i, 0)),
                 pl.BlockSpec((RB, 128), lambda i: (i, 0))],
      out_shape=[jax.ShapeDtypeStruct((VOCAB, 256), jnp.float32),
                 jax.ShapeDtypeStruct((VOCAB, 128), jnp.float32)],
  )(xt)


def _mlp_body(a0, a1, a2, a3, w1_ref, b1_ref, w2_ref, b2_ref, out_ref):
  h = None
  for t, a in enumerate((a0, a1, a2, a3)):
    p = lax.dot_general(a[...], w1_ref[pl.ds(EPAD * t, EPAD), :],
                        (((1,), (0,)), ((), ())),
                        preferred_element_type=jnp.float32,
                        precision=lax.Precision.HIGHEST)
    h = p if h is None else h + p
  h = h * (1.0 / SEQ_LEN) + b1_ref[...]
  h = jnp.maximum(h, 0.0)
  logits = lax.dot_general(h, w2_ref[...], (((1,), (0,)), ((), ())),
                           preferred_element_type=jnp.float32,
                           precision=lax.Precision.HIGHEST) + b2_ref[...]
  m = jnp.max(logits, axis=1, keepdims=True)
  lse = jnp.log(jnp.sum(jnp.exp(logits - m), axis=1, keepdims=True)) + m
  out_ref[...] = logits - lse


def kernel(text, entity1, text_emb, entity_emb, ngram2_emb, ngram3_emb,
           W1, b1, W2, b2):
  bt = jnp.pad(text.T, ((0, 0), (0, LPAD - SEQ_LEN)))
  be = jnp.pad(entity1.T, ((0, 0), (0, LPAD - SEQ_LEN)))

  m_e, tl_e = _relayout(entity_emb.T)
  t_gate, _ = lax.optimization_barrier((text_emb, tl_e))
  m_t, tl_t = _relayout(t_gate.T)
  n2_gate, _ = lax.optimization_barrier((ngram2_emb, tl_t))
  m_n2, tl_n2 = _relayout(n2_gate.T)
  n3_gate, _ = lax.optimization_barrier((ngram3_emb, tl_n2))
  m_n3, tl_n3 = _relayout(n3_gate.T)

  pool = _make_sc_pool(VOCAB)
  acc_e = pool(be, m_e, tl_e)
  acc_t = pool(bt, m_t, tl_t)
  acc_n2 = pool(bt, m_n2, tl_n2)
  acc_n3 = pool(bt, m_n3, tl_n3)
  accs = [a.reshape(BATCH, EPAD) for a in (acc_e, acc_t, acc_n2, acc_n3)]

  w1r = W1.reshape(NUM_TABLES, EMBED, HIDDEN)
  w1p = jnp.zeros((NUM_TABLES, EPAD, HIDDEN), jnp.float32)
  w1p = w1p.at[:, :EMBED, :].set(w1r)
  w1p = w1p.reshape(NUM_TABLES * EPAD, HIDDEN)

  return pl.pallas_call(
      _mlp_body,
      out_shape=jax.ShapeDtypeStruct((BATCH, NUM_CLASSES), jnp.float32),
  )(*accs, w1p, b1.reshape(1, HIDDEN), W2, b2.reshape(1, NUM_CLASSES))


# speedup vs baseline: 1.8680x; 1.0488x over previous
import jax
import jax.numpy as jnp
from jax import lax
from jax.experimental import pallas as pl
from jax.experimental.pallas import tpu as pltpu
from jax.experimental.pallas import tpu_sc as plsc

VOCAB = 100000
EMBED = 300
SEQ_LEN = 200
BATCH = 1024
HIDDEN = 256
NUM_CLASSES = 2
NUM_TABLES = 4

NUM_CORES = 2
NUM_SUBCORES = 16
NUM_WORKERS = NUM_CORES * NUM_SUBCORES
SPW = BATCH // NUM_WORKERS

LPAD = 256
TAIL = EMBED - 256
EPAD = 384


def _sc_table_body(idx_hbm, tbl, tail, out_hbm, idx8, buf, outb8,
                   semg0, semg1):
  semg = (semg0, semg1)
  wid = lax.axis_index("s") * NUM_CORES + lax.axis_index("c")
  base = wid * SPW

  zero16 = jnp.zeros((16,), jnp.float32)
  def zbody(q, carry):
    for m in range(8):
      outb8[q // 8, q % 8, pl.ds(16 * m, 16)] = zero16
    return carry
  lax.fori_loop(0, 3 * 8, zbody, 0)

  segs = (pl.ds(0, 128), pl.ds(128, 72))

  def step_copy_args(st, s, sl):
    if st < 2:
      return [(tbl.at[idx8.at[s, seg], pl.ds(128 * st, 128)],
               buf.at[sl, seg, :], semg[sl]) for seg in segs]
    return [(tail.at[idx8.at[s, seg], :], buf.at[sl, seg, :], semg[sl])
            for seg in segs]

  def issue_step(st, s, sl):
    for a in step_copy_args(st, s, sl):
      pltpu.async_copy(*a)

  def wait_step(st, s, sl):
    for a in step_copy_args(st, s, sl):
      pltpu.make_async_copy(*a).wait()

  def reduce_step(st, s, sl):
    cols = tuple(16 * m for m in range(8)) if st < 2 else (0, 16, 28)
    def rbody(r, accs):
      return tuple(acc + buf[sl, r, pl.ds(c, 16)]
                   for acc, c in zip(accs, cols))
    init = tuple(buf[sl, 0, pl.ds(c, 16)] for c in cols)
    accs = lax.fori_loop(1, SEQ_LEN, rbody, init)
    for acc, c in zip(accs, cols):
      outb8[st, s, pl.ds(c, 16)] = acc

  def do_pair(k, carry):
    i0 = 2 * k
    s0 = lax.rem(i0, 8)
    b0 = base + i0

    @pl.when(s0 == 0)
    def _():
      bg = pl.multiple_of(b0, 8)
      pltpu.sync_copy(idx_hbm.at[pl.ds(bg, 8)], idx8)
      issue_step(0, s0, 0)

    for half in range(2):
      s = s0 + half
      for st in range(3):
        sl = (3 * half + st) % 2
        nsl = 1 - sl
        if st < 2:
          issue_step(st + 1, s, nsl)
        elif half == 0:
          issue_step(0, s + 1, nsl)
        else:
          @pl.when(lax.rem(s, 8) < 7)
          def _():
            issue_step(0, s + 1, nsl)
        wait_step(st, s, sl)
        reduce_step(st, s, sl)

      @pl.when(lax.rem(s, 8) == 7)
      def _():
        grp = (base + i0 + half - 7) // 8
        pltpu.sync_copy(outb8, out_hbm.at[grp])
    return carry

  lax.fori_loop(0, SPW // 2, do_pair, 0)


def _make_sc_pool(vocab):
  mesh = plsc.VectorSubcoreMesh(core_axis_name="c", subcore_axis_name="s",
                                num_cores=NUM_CORES,
                                num_subcores=NUM_SUBCORES)
  return pl.kernel(
      _sc_table_body,
      out_type=jax.ShapeDtypeStruct((BATCH // 8, 3, 8, 128), jnp.float32),
      mesh=mesh,
      scratch_types=[
          pltpu.VMEM((8, LPAD), jnp.int32),
          pltpu.VMEM((2, SEQ_LEN, 128), jnp.float32),
          pltpu.VMEM((3, 8, 128), jnp.float32),
          pltpu.SemaphoreType.DMA,
          pltpu.SemaphoreType.DMA,
      ],
  )


RB = 1024


def _relayout_body(xt_ref, om_ref, ot_ref):
  x = xt_ref[...].T
  om_ref[...] = x[:, :256]
  ot_ref[:, :TAIL] = x[:, 256:]


def _relayout(xt):
  grid = ((VOCAB + RB - 1) // RB,)
  return pl.pallas_call(
      _relayout_body,
      grid=grid,
      in_specs=[pl.BlockSpec((EMBED, RB), lambda i: (0, i))],
      out_specs=[pl.BlockSpec((RB, 256), lambda i: (i, 0)),
                 pl.BlockSpec((RB, 128), lambda i: (i, 0))],
      out_shape=[jax.ShapeDtypeStruct((VOCAB, 256), jnp.float32),
                 jax.ShapeDtypeStruct((VOCAB, 128), jnp.float32)],
  )(xt)


def _mlp_body(a0, a1, a2, a3, w1_ref, b1_ref, w2_ref, b2_ref, out_ref):
  h = None
  for t, a in enumerate((a0, a1, a2, a3)):
    p = lax.dot_general(a[...], w1_ref[pl.ds(EPAD * t, EPAD), :],
                        (((1,), (0,)), ((), ())),
                        preferred_element_type=jnp.float32,
                        precision=lax.Precision.HIGHEST)
    h = p if h is None else h + p
  h = h * (1.0 / SEQ_LEN) + b1_ref[...]
  h = jnp.maximum(h, 0.0)
  logits = lax.dot_general(h, w2_ref[...], (((1,), (0,)), ((), ())),
                           preferred_element_type=jnp.float32,
                           precision=lax.Precision.HIGHEST) + b2_ref[...]
  m = jnp.max(logits, axis=1, keepdims=True)
  lse = jnp.log(jnp.sum(jnp.exp(logits - m), axis=1, keepdims=True)) + m
  out_ref[...] = logits - lse


def kernel(text, entity1, text_emb, entity_emb, ngram2_emb, ngram3_emb,
           W1, b1, W2, b2):
  bt = jnp.pad(text.T, ((0, 0), (0, LPAD - SEQ_LEN)))
  be = jnp.pad(entity1.T, ((0, 0), (0, LPAD - SEQ_LEN)))

  m_e, tl_e = _relayout(entity_emb.T)
  t_gate, _ = lax.optimization_barrier((text_emb, tl_e))
  m_t, tl_t = _relayout(t_gate.T)
  n2_gate, _ = lax.optimization_barrier((ngram2_emb, tl_t))
  m_n2, tl_n2 = _relayout(n2_gate.T)
  n3_gate, _ = lax.optimization_barrier((ngram3_emb, tl_n2))
  m_n3, tl_n3 = _relayout(n3_gate.T)

  pool = _make_sc_pool(VOCAB)
  acc_e = pool(be, m_e, tl_e)
  acc_t = pool(bt, m_t, tl_t)
  acc_n2 = pool(bt, m_n2, tl_n2)
  acc_n3 = pool(bt, m_n3, tl_n3)
  accs = [a.reshape(BATCH, EPAD) for a in (acc_e, acc_t, acc_n2, acc_n3)]

  w1r = W1.reshape(NUM_TABLES, EMBED, HIDDEN)
  w1p = jnp.zeros((NUM_TABLES, EPAD, HIDDEN), jnp.float32)
  w1p = w1p.at[:, :EMBED, :].set(w1r)
  w1p = w1p.reshape(NUM_TABLES * EPAD, HIDDEN)

  return pl.pallas_call(
      _mlp_body,
      out_shape=jax.ShapeDtypeStruct((BATCH, NUM_CLASSES), jnp.float32),
  )(*accs, w1p, b1.reshape(1, HIDDEN), W2, b2.reshape(1, NUM_CLASSES))
